# baseline (speedup 1.0000x reference)
import jax
import jax.numpy as jnp
from jax import lax
from jax.experimental import pallas as pl
from jax.experimental.pallas import tpu as pltpu, tpu_sc as plsc

N = 10000
E = 320000
DIN = 128
DH = 256

NB = 79
N_PAD = NB * 128
NT = 16
NR = N_PAD // NT
CH = 128
CHUNKS_FULL = 160
CHUNKS_HALF = 80
E_PAD = CHUNKS_FULL * NT * CH
TOTCH = E_PAD // CH

_MESH = plsc.VectorSubcoreMesh(core_axis_name="c", subcore_axis_name="s")


def _deg_body(edges_hbm, ones_hbm, zeros_hbm, out_hbm, idx_v, ones_v, deg_sh):
    c = lax.axis_index("c")
    s = lax.axis_index("s")
    r0 = s * NR
    pltpu.sync_copy(zeros_hbm.at[pl.ds(r0, NR)], deg_sh.at[pl.ds(r0, NR)])
    pltpu.sync_copy(ones_hbm, ones_v)
    plsc.subcore_barrier()
    ebase = c * E_PAD + s * (CHUNKS_FULL * CH)

    def chunk(k, carry):
        b = ebase + k * CH
        pltpu.sync_copy(edges_hbm.at[pl.ds(b, CH)], idx_v)
        pltpu.sync_copy(ones_v, deg_sh.at[idx_v], add=True)
        return carry

    lax.fori_loop(0, CHUNKS_FULL, chunk, 0)
    plsc.subcore_barrier()
    pltpu.sync_copy(deg_sh.at[pl.ds(r0, NR)],
                    out_hbm.at[pl.ds(c * N_PAD + r0, NR)])


_deg_call = pl.kernel(
    _deg_body,
    out_type=jax.ShapeDtypeStruct((2 * N_PAD, 128), jnp.float32),
    mesh=_MESH,
    scratch_types=[
        pltpu.VMEM((CH,), jnp.int32),
        pltpu.VMEM((CH, 128), jnp.float32),
        pltpu.VMEM_SHARED((N_PAD, 128), jnp.float32),
    ],
)


def _spmm_body(col_split, src_hbm, dst_hbm, table_hbm, zeros_hbm, out_hbm,
               idxs, idxo, idxd, rows_v, agg_sh, sem):
    c = lax.axis_index("c")
    s = lax.axis_index("s")
    r0 = s * NR
    pltpu.sync_copy(zeros_hbm.at[pl.ds(r0, NR)], agg_sh.at[pl.ds(r0, NR)])
    plsc.subcore_barrier()
    if col_split:
        chunks = CHUNKS_FULL
        ebase = s * (CHUNKS_FULL * CH)
        off = c * N_PAD
    else:
        chunks = CHUNKS_HALF
        ebase = (c * NT + s) * (CHUNKS_HALF * CH)
        off = None

    def chunk(k, carry):
        b = ebase + k * CH
        pltpu.sync_copy(src_hbm.at[pl.ds(b, CH)], idxs)
        if off is not None:
            for j in range(CH // 16):
                idxo[pl.ds(j * 16, 16)] = idxs[pl.ds(j * 16, 16)] + off
            gidx = idxo
        else:
            gidx = idxs
        pltpu.async_copy(table_hbm.at[gidx], rows_v, sem).wait()
        pltpu.sync_copy(dst_hbm.at[pl.ds(b, CH)], idxd)
        pltpu.sync_copy(rows_v, agg_sh.at[idxd], add=True)
        return carry

    lax.fori_loop(0, chunks, chunk, 0)
    plsc.subcore_barrier()
    pltpu.sync_copy(agg_sh.at[pl.ds(r0, NR)],
                    out_hbm.at[pl.ds(c * N_PAD + r0, NR)])


def _make_spmm(col_split):
    return pl.kernel(
        lambda *args: _spmm_body(col_split, *args),
        out_type=jax.ShapeDtypeStruct((2 * N_PAD, 128), jnp.float32),
        mesh=_MESH,
        scratch_types=[
            pltpu.VMEM((CH,), jnp.int32),
            pltpu.VMEM((CH,), jnp.int32),
            pltpu.VMEM((CH,), jnp.int32),
            pltpu.VMEM((CH, 128), jnp.float32),
            pltpu.VMEM_SHARED((N_PAD, 128), jnp.float32),
            pltpu.SemaphoreType.DMA,
        ],
    )


_spmm_l1 = _make_spmm(False)
_spmm_l2 = _make_spmm(True)


def _tc1_body(x_ref, dego_ref, xs_ref):
    sc = lax.rsqrt(jnp.maximum(dego_ref[:, 0:1], 1.0))
    xs_ref[...] = x_ref[...] * sc


_tc1_call = pl.pallas_call(
    _tc1_body,
    grid=(NB,),
    in_specs=[
        pl.BlockSpec((128, 128), lambda i: (i, 0)),
        pl.BlockSpec((128, 128), lambda i: (i, 0)),
    ],
    out_specs=pl.BlockSpec((128, 128), lambda i: (i, 0)),
    out_shape=jax.ShapeDtypeStruct((N_PAD, 128), jnp.float32),
)


def _tc2_body(agga_ref, aggb_ref, dego_ref, degi_ref, w_ref, b_ref, out_ref):
    a = agga_ref[...] + aggb_ref[...]
    t = lax.rsqrt(jnp.maximum(degi_ref[:, 0:1], 1.0))
    y = jnp.dot(t * a, w_ref[...], preferred_element_type=jnp.float32)
    y = y + b_ref[...]
    y = jnp.where(y > 0, y, 0.01 * y)
    sc = lax.rsqrt(jnp.maximum(dego_ref[:, 0:1], 1.0))
    out_ref[...] = sc * y


_tc2_call = pl.pallas_call(
    _tc2_body,
    grid=(2, NB),
    in_specs=[
        pl.BlockSpec((128, 128), lambda j, i: (i, 0)),
        pl.BlockSpec((128, 128), lambda j, i: (NB + i, 0)),
        pl.BlockSpec((128, 128), lambda j, i: (i, 0)),
        pl.BlockSpec((128, 128), lambda j, i: (NB + i, 0)),
        pl.BlockSpec((128, 128), lambda j, i: (0, j)),
        pl.BlockSpec((1, 128), lambda j, i: (0, j)),
    ],
    out_specs=pl.BlockSpec((128, 128), lambda j, i: (j * NB + i, 0)),
    out_shape=jax.ShapeDtypeStruct((2 * N_PAD, 128), jnp.float32),
)


def _tc3_body(agga_ref, aggb_ref, degi_ref, w_ref, b_ref, out_ref):
    a = jnp.concatenate([agga_ref[...], aggb_ref[...]], axis=1)
    t = lax.rsqrt(jnp.maximum(degi_ref[:, 0:1], 1.0))
    y = jnp.dot(t * a, w_ref[...], preferred_element_type=jnp.float32)
    out_ref[...] = y + b_ref[...]


_tc3_call = pl.pallas_call(
    _tc3_body,
    grid=(2, NB),
    in_specs=[
        pl.BlockSpec((128, 128), lambda j, i: (i, 0)),
        pl.BlockSpec((128, 128), lambda j, i: (NB + i, 0)),
        pl.BlockSpec((128, 128), lambda j, i: (NB + i, 0)),
        pl.BlockSpec((256, 128), lambda j, i: (0, j)),
        pl.BlockSpec((1, 128), lambda j, i: (0, j)),
    ],
    out_specs=pl.BlockSpec((128, 128), lambda j, i: (i, j)),
    out_shape=jax.ShapeDtypeStruct((N_PAD, DH), jnp.float32),
)


def kernel(n_feat, edge_index, W1, b1, W2, b2):
    f32 = jnp.float32
    x_pad = jnp.zeros((N_PAD, DIN), f32).at[:N].set(n_feat)
    src_pad = jnp.full((E_PAD,), N, jnp.int32).at[:E].set(edge_index[0])
    dst_pad = jnp.full((E_PAD,), N, jnp.int32).at[:E].set(edge_index[1])
    edges_flat = jnp.concatenate([src_pad, dst_pad])
    ones128 = jnp.ones((CH, 128), f32)
    zeros128 = jnp.zeros((N_PAD, 128), f32)

    degs = _deg_call(edges_flat, ones128, zeros128)
    xs1 = _tc1_call(x_pad, degs)
    agg1 = _spmm_l1(src_pad, dst_pad, xs1, zeros128)
    xs2 = _tc2_call(agg1, agg1, degs, degs, W1, b1.reshape(1, DH))
    agg2 = _spmm_l2(src_pad, dst_pad, xs2, zeros128)
    out = _tc3_call(agg2, agg2, degs, W2, b2.reshape(1, DH))
    return out[:N]


# speedup vs baseline: 1.4963x; 1.4963x over previous
import jax
import jax.numpy as jnp
from jax import lax
from jax.experimental import pallas as pl
from jax.experimental.pallas import tpu as pltpu, tpu_sc as plsc

N = 10000
E = 320000
DIN = 128
DH = 256

NB = 79
N_PAD = NB * 128
NT = 16
NR = N_PAD // NT
CH = 128
CHUNKS_FULL = 160
CHUNKS_HALF = 80
E_PAD = CHUNKS_FULL * NT * CH
TOTCH = E_PAD // CH

_MESH = plsc.VectorSubcoreMesh(core_axis_name="c", subcore_axis_name="s")


def _deg_body(edges_hbm, ones_hbm, zeros_hbm, out_hbm, idx_v, ones_v, deg_sh):
    c = lax.axis_index("c")
    s = lax.axis_index("s")
    r0 = s * NR
    pltpu.sync_copy(zeros_hbm.at[pl.ds(r0, NR)], deg_sh.at[pl.ds(r0, NR)])
    pltpu.sync_copy(ones_hbm, ones_v)
    plsc.subcore_barrier()
    ebase = c * E_PAD + s * (CHUNKS_FULL * CH)

    def chunk(k, carry):
        b = ebase + k * CH
        pltpu.sync_copy(edges_hbm.at[pl.ds(b, CH)], idx_v)
        pltpu.sync_copy(ones_v, deg_sh.at[idx_v], add=True)
        return carry

    lax.fori_loop(0, CHUNKS_FULL, chunk, 0)
    plsc.subcore_barrier()
    pltpu.sync_copy(deg_sh.at[pl.ds(r0, NR)],
                    out_hbm.at[pl.ds(c * N_PAD + r0, NR)])


_deg_call = pl.kernel(
    _deg_body,
    out_type=jax.ShapeDtypeStruct((2 * N_PAD, 128), jnp.float32),
    mesh=_MESH,
    scratch_types=[
        pltpu.VMEM((CH,), jnp.int32),
        pltpu.VMEM((CH, 128), jnp.float32),
        pltpu.VMEM_SHARED((N_PAD, 128), jnp.float32),
    ],
)


def _spmm_body(col_split, src_hbm, dst_hbm, table_hbm, zeros_hbm, out_hbm,
               idxs, idxo, idxd, rows_v, agg_sh, sem):
    c = lax.axis_index("c")
    s = lax.axis_index("s")
    r0 = s * NR
    pltpu.sync_copy(zeros_hbm.at[pl.ds(r0, NR)], agg_sh.at[pl.ds(r0, NR)])
    plsc.subcore_barrier()
    if col_split:
        chunks = CHUNKS_FULL
        ebase = s * (CHUNKS_FULL * CH)
        off = c * N_PAD
    else:
        chunks = CHUNKS_HALF
        ebase = (c * NT + s) * (CHUNKS_HALF * CH)
        off = None

    def chunk(k, carry):
        b = ebase + k * CH
        pltpu.sync_copy(src_hbm.at[pl.ds(b, CH)], idxs)
        if off is not None:
            for j in range(CH // 16):
                idxo[pl.ds(j * 16, 16)] = idxs[pl.ds(j * 16, 16)] + off
            gidx = idxo
        else:
            gidx = idxs
        pltpu.async_copy(table_hbm.at[gidx], rows_v, sem).wait()
        pltpu.sync_copy(dst_hbm.at[pl.ds(b, CH)], idxd)
        pltpu.sync_copy(rows_v, agg_sh.at[idxd], add=True)
        return carry

    lax.fori_loop(0, chunks, chunk, 0)
    plsc.subcore_barrier()
    pltpu.sync_copy(agg_sh.at[pl.ds(r0, NR)],
                    out_hbm.at[pl.ds(c * N_PAD + r0, NR)])


def _make_spmm(col_split):
    return pl.kernel(
        lambda *args: _spmm_body(col_split, *args),
        out_type=jax.ShapeDtypeStruct((2 * N_PAD, 128), jnp.float32),
        mesh=_MESH,
        scratch_types=[
            pltpu.VMEM((CH,), jnp.int32),
            pltpu.VMEM((CH,), jnp.int32),
            pltpu.VMEM((CH,), jnp.int32),
            pltpu.VMEM((CH, 128), jnp.float32),
            pltpu.VMEM_SHARED((N_PAD, 128), jnp.float32),
            pltpu.SemaphoreType.DMA,
        ],
    )


_spmm_l1 = _make_spmm(False)
_spmm_l2 = _make_spmm(True)


def _tc1_body(x_ref, dego_ref, xs_ref):
    sc = lax.rsqrt(jnp.maximum(dego_ref[:, 0:1], 1.0))
    xs_ref[...] = x_ref[...] * sc


_tc1_call = pl.pallas_call(
    _tc1_body,
    grid=(NB,),
    in_specs=[
        pl.BlockSpec((128, 128), lambda i: (i, 0)),
        pl.BlockSpec((128, 128), lambda i: (i, 0)),
    ],
    out_specs=pl.BlockSpec((128, 128), lambda i: (i, 0)),
    out_shape=jax.ShapeDtypeStruct((N_PAD, 128), jnp.float32),
)


def _tc2_body(agga_ref, aggb_ref, dego_ref, degi_ref, w_ref, b_ref, out_ref):
    a = agga_ref[...] + aggb_ref[...]
    t = lax.rsqrt(jnp.maximum(degi_ref[:, 0:1], 1.0))
    y = jnp.dot(t * a, w_ref[...], preferred_element_type=jnp.float32)
    y = y + b_ref[...]
    y = jnp.where(y > 0, y, 0.01 * y)
    sc = lax.rsqrt(jnp.maximum(dego_ref[:, 0:1], 1.0))
    out_ref[...] = sc * y


_tc2_call = pl.pallas_call(
    _tc2_body,
    grid=(2, NB),
    in_specs=[
        pl.BlockSpec((128, 128), lambda j, i: (i, 0)),
        pl.BlockSpec((128, 128), lambda j, i: (NB + i, 0)),
        pl.BlockSpec((128, 128), lambda j, i: (i, 0)),
        pl.BlockSpec((128, 128), lambda j, i: (NB + i, 0)),
        pl.BlockSpec((128, 128), lambda j, i: (0, j)),
        pl.BlockSpec((1, 128), lambda j, i: (0, j)),
    ],
    out_specs=pl.BlockSpec((128, 128), lambda j, i: (j * NB + i, 0)),
    out_shape=jax.ShapeDtypeStruct((2 * N_PAD, 128), jnp.float32),
)


def _tc3_body(agga_ref, aggb_ref, degi_ref, w_ref, b_ref, out_ref):
    a = jnp.concatenate([agga_ref[...], aggb_ref[...]], axis=1)
    t = lax.rsqrt(jnp.maximum(degi_ref[:, 0:1], 1.0))
    y = jnp.dot(t * a, w_ref[...], preferred_element_type=jnp.float32)
    out_ref[...] = y + b_ref[...]


_tc3_call = pl.pallas_call(
    _tc3_body,
    grid=(2, NB),
    in_specs=[
        pl.BlockSpec((128, 128), lambda j, i: (i, 0)),
        pl.BlockSpec((128, 128), lambda j, i: (NB + i, 0)),
        pl.BlockSpec((128, 128), lambda j, i: (NB + i, 0)),
        pl.BlockSpec((256, 128), lambda j, i: (0, j)),
        pl.BlockSpec((1, 128), lambda j, i: (0, j)),
    ],
    out_specs=pl.BlockSpec((128, 128), lambda j, i: (i, j)),
    out_shape=jax.ShapeDtypeStruct((N_PAD, DH), jnp.float32),
)


def kernel(n_feat, edge_index, W1, b1, W2, b2):
    f32 = jnp.float32
    x_pad = jnp.zeros((N_PAD, DIN), f32).at[:N].set(n_feat)
    pad_idx = N + jnp.arange(E_PAD, dtype=jnp.int32) % (N_PAD - N)
    src_pad = pad_idx.at[:E].set(edge_index[0])
    dst_pad = pad_idx.at[:E].set(edge_index[1])
    edges_flat = jnp.concatenate([src_pad, dst_pad])
    ones128 = jnp.ones((CH, 128), f32)
    zeros128 = jnp.zeros((N_PAD, 128), f32)

    degs = _deg_call(edges_flat, ones128, zeros128)
    xs1 = _tc1_call(x_pad, degs)
    agg1 = _spmm_l1(src_pad, dst_pad, xs1, zeros128)
    xs2 = _tc2_call(agg1, agg1, degs, degs, W1, b1.reshape(1, DH))
    agg2 = _spmm_l2(src_pad, dst_pad, xs2, zeros128)
    out = _tc3_call(agg2, agg2, degs, W2, b2.reshape(1, DH))
    return out[:N]


# speedup vs baseline: 1.6996x; 1.1359x over previous
import jax
import jax.numpy as jnp
from jax import lax
from jax.experimental import pallas as pl
from jax.experimental.pallas import tpu as pltpu, tpu_sc as plsc

N = 10000
E = 320000
DIN = 128
DH = 256

NB = 79
N_PAD = NB * 128
NT = 16
NR = N_PAD // NT
CH = 128
CHUNKS_FULL = 160
CHUNKS_HALF = 80
E_PAD = CHUNKS_FULL * NT * CH
TOTCH = E_PAD // CH

_MESH = plsc.VectorSubcoreMesh(core_axis_name="c", subcore_axis_name="s")


def _deg_body(edges_hbm, ones_hbm, zeros_hbm, out_hbm, idx_v, ones_v, deg_sh):
    c = lax.axis_index("c")
    s = lax.axis_index("s")
    r0 = s * NR
    pltpu.sync_copy(zeros_hbm.at[pl.ds(r0, NR)], deg_sh.at[pl.ds(r0, NR)])
    pltpu.sync_copy(ones_hbm, ones_v)
    plsc.subcore_barrier()
    ebase = c * E_PAD + s * (CHUNKS_FULL * CH)

    def chunk(k, carry):
        b = ebase + k * CH
        pltpu.sync_copy(edges_hbm.at[pl.ds(b, CH)], idx_v)
        pltpu.sync_copy(ones_v, deg_sh.at[idx_v], add=True)
        return carry

    lax.fori_loop(0, CHUNKS_FULL, chunk, 0)
    plsc.subcore_barrier()
    pltpu.sync_copy(deg_sh.at[pl.ds(r0, NR)],
                    out_hbm.at[pl.ds(c * N_PAD + r0, NR)])


_deg_call = pl.kernel(
    _deg_body,
    out_type=jax.ShapeDtypeStruct((2 * N_PAD, 128), jnp.float32),
    mesh=_MESH,
    scratch_types=[
        pltpu.VMEM((CH,), jnp.int32),
        pltpu.VMEM((CH, 128), jnp.float32),
        pltpu.VMEM_SHARED((N_PAD, 128), jnp.float32),
    ],
)


def _spmm_body(col_split, src_hbm, dst_hbm, table_hbm, zeros_hbm, out_hbm,
               idxs, idxo, idxd2, rows2, agg_sh, gsem, ssem0, ssem1):
    c = lax.axis_index("c")
    s = lax.axis_index("s")
    r0 = s * NR
    pltpu.sync_copy(zeros_hbm.at[pl.ds(r0, NR)], agg_sh.at[pl.ds(r0, NR)])
    plsc.subcore_barrier()
    ssem = [ssem0, ssem1]
    if col_split:
        chunks = CHUNKS_FULL
        ebase = s * (CHUNKS_FULL * CH)
        off = c * N_PAD
    else:
        chunks = CHUNKS_HALF
        ebase = (c * NT + s) * (CHUNKS_HALF * CH)
        off = None

    def window(w, carry):
        for b2 in range(2):
            k = 2 * w + b2
            b = ebase + k * CH

            @pl.when(k >= 2)
            def _():
                pltpu.make_async_copy(zeros_hbm.at[pl.ds(0, CH)],
                                      rows2.at[b2], ssem[b2]).wait()

            pltpu.sync_copy(src_hbm.at[pl.ds(b, CH)], idxs)
            if off is not None:
                for j in range(CH // 16):
                    idxo[pl.ds(j * 16, 16)] = idxs[pl.ds(j * 16, 16)] + off
                gidx = idxo
            else:
                gidx = idxs
            pltpu.async_copy(table_hbm.at[gidx], rows2.at[b2], gsem).wait()
            pltpu.sync_copy(dst_hbm.at[pl.ds(b, CH)], idxd2.at[b2])
            pltpu.async_copy(rows2.at[b2], agg_sh.at[idxd2.at[b2]],
                             ssem[b2], add=True)
        return carry

    lax.fori_loop(0, chunks // 2, window, 0)
    pltpu.make_async_copy(zeros_hbm.at[pl.ds(0, CH)], rows2.at[0],
                          ssem[0]).wait()
    pltpu.make_async_copy(zeros_hbm.at[pl.ds(0, CH)], rows2.at[1],
                          ssem[1]).wait()
    plsc.subcore_barrier()
    pltpu.sync_copy(agg_sh.at[pl.ds(r0, NR)],
                    out_hbm.at[pl.ds(c * N_PAD + r0, NR)])


def _make_spmm(col_split):
    return pl.kernel(
        lambda *args: _spmm_body(col_split, *args),
        out_type=jax.ShapeDtypeStruct((2 * N_PAD, 128), jnp.float32),
        mesh=_MESH,
        scratch_types=[
            pltpu.VMEM((CH,), jnp.int32),
            pltpu.VMEM((CH,), jnp.int32),
            pltpu.VMEM((2, CH), jnp.int32),
            pltpu.VMEM((2, CH, 128), jnp.float32),
            pltpu.VMEM_SHARED((N_PAD, 128), jnp.float32),
            pltpu.SemaphoreType.DMA,
            pltpu.SemaphoreType.DMA,
            pltpu.SemaphoreType.DMA,
        ],
    )


_spmm_l1 = _make_spmm(False)
_spmm_l2 = _make_spmm(True)


def _tc1_body(x_ref, dego_ref, xs_ref):
    sc = lax.rsqrt(jnp.maximum(dego_ref[:, 0:1], 1.0))
    xs_ref[...] = x_ref[...] * sc


_tc1_call = pl.pallas_call(
    _tc1_body,
    grid=(NB,),
    in_specs=[
        pl.BlockSpec((128, 128), lambda i: (i, 0)),
        pl.BlockSpec((128, 128), lambda i: (i, 0)),
    ],
    out_specs=pl.BlockSpec((128, 128), lambda i: (i, 0)),
    out_shape=jax.ShapeDtypeStruct((N_PAD, 128), jnp.float32),
)


def _tc2_body(agga_ref, aggb_ref, dego_ref, degi_ref, w_ref, b_ref, out_ref):
    a = agga_ref[...] + aggb_ref[...]
    t = lax.rsqrt(jnp.maximum(degi_ref[:, 0:1], 1.0))
    y = jnp.dot(t * a, w_ref[...], preferred_element_type=jnp.float32)
    y = y + b_ref[...]
    y = jnp.where(y > 0, y, 0.01 * y)
    sc = lax.rsqrt(jnp.maximum(dego_ref[:, 0:1], 1.0))
    out_ref[...] = sc * y


_tc2_call = pl.pallas_call(
    _tc2_body,
    grid=(2, NB),
    in_specs=[
        pl.BlockSpec((128, 128), lambda j, i: (i, 0)),
        pl.BlockSpec((128, 128), lambda j, i: (NB + i, 0)),
        pl.BlockSpec((128, 128), lambda j, i: (i, 0)),
        pl.BlockSpec((128, 128), lambda j, i: (NB + i, 0)),
        pl.BlockSpec((128, 128), lambda j, i: (0, j)),
        pl.BlockSpec((1, 128), lambda j, i: (0, j)),
    ],
    out_specs=pl.BlockSpec((128, 128), lambda j, i: (j * NB + i, 0)),
    out_shape=jax.ShapeDtypeStruct((2 * N_PAD, 128), jnp.float32),
)


def _tc3_body(agga_ref, aggb_ref, degi_ref, w_ref, b_ref, out_ref):
    a = jnp.concatenate([agga_ref[...], aggb_ref[...]], axis=1)
    t = lax.rsqrt(jnp.maximum(degi_ref[:, 0:1], 1.0))
    y = jnp.dot(t * a, w_ref[...], preferred_element_type=jnp.float32)
    out_ref[...] = y + b_ref[...]


_tc3_call = pl.pallas_call(
    _tc3_body,
    grid=(2, NB),
    in_specs=[
        pl.BlockSpec((128, 128), lambda j, i: (i, 0)),
        pl.BlockSpec((128, 128), lambda j, i: (NB + i, 0)),
        pl.BlockSpec((128, 128), lambda j, i: (NB + i, 0)),
        pl.BlockSpec((256, 128), lambda j, i: (0, j)),
        pl.BlockSpec((1, 128), lambda j, i: (0, j)),
    ],
    out_specs=pl.BlockSpec((128, 128), lambda j, i: (i, j)),
    out_shape=jax.ShapeDtypeStruct((N_PAD, DH), jnp.float32),
)


def kernel(n_feat, edge_index, W1, b1, W2, b2):
    f32 = jnp.float32
    x_pad = jnp.zeros((N_PAD, DIN), f32).at[:N].set(n_feat)
    pad_idx = N + jnp.arange(E_PAD, dtype=jnp.int32) % (N_PAD - N)
    src_pad = pad_idx.at[:E].set(edge_index[0])
    dst_pad = pad_idx.at[:E].set(edge_index[1])
    edges_flat = jnp.concatenate([src_pad, dst_pad])
    ones128 = jnp.ones((CH, 128), f32)
    zeros128 = jnp.zeros((N_PAD, 128), f32)

    degs = _deg_call(edges_flat, ones128, zeros128)
    xs1 = _tc1_call(x_pad, degs)
    agg1 = _spmm_l1(src_pad, dst_pad, xs1, zeros128)
    xs2 = _tc2_call(agg1, agg1, degs, degs, W1, b1.reshape(1, DH))
    agg2 = _spmm_l2(src_pad, dst_pad, xs2, zeros128)
    out = _tc3_call(agg2, agg2, degs, W2, b2.reshape(1, DH))
    return out[:N]


# speedup vs baseline: 1.8709x; 1.1008x over previous
import jax
import jax.numpy as jnp
from jax import lax
from jax.experimental import pallas as pl
from jax.experimental.pallas import tpu as pltpu, tpu_sc as plsc

N = 10000
E = 320000
DIN = 128
DH = 256

NB = 79
N_PAD = NB * 128
NT = 16
NR = N_PAD // NT
CH = 128
CHUNKS_FULL = 160
CHUNKS_HALF = 80
E_PAD = CHUNKS_FULL * NT * CH
TOTCH = E_PAD // CH

_MESH = plsc.VectorSubcoreMesh(core_axis_name="c", subcore_axis_name="s")


def _deg_body(edges_hbm, ones_hbm, zeros_hbm, out_hbm, idx_v, ones_v, deg_sh):
    c = lax.axis_index("c")
    s = lax.axis_index("s")
    r0 = s * NR
    pltpu.sync_copy(zeros_hbm.at[pl.ds(r0, NR)], deg_sh.at[pl.ds(r0, NR)])
    pltpu.sync_copy(ones_hbm, ones_v)
    plsc.subcore_barrier()
    ebase = c * E_PAD + s * (CHUNKS_FULL * CH)

    def chunk(k, carry):
        b = ebase + k * CH
        pltpu.sync_copy(edges_hbm.at[pl.ds(b, CH)], idx_v)
        pltpu.sync_copy(ones_v, deg_sh.at[idx_v], add=True)
        return carry

    lax.fori_loop(0, CHUNKS_FULL, chunk, 0)
    plsc.subcore_barrier()
    pltpu.sync_copy(deg_sh.at[pl.ds(r0, NR)],
                    out_hbm.at[pl.ds(c * N_PAD + r0, NR)])


_deg_call = pl.kernel(
    _deg_body,
    out_type=jax.ShapeDtypeStruct((2 * N_PAD, 128), jnp.float32),
    mesh=_MESH,
    scratch_types=[
        pltpu.VMEM((CH,), jnp.int32),
        pltpu.VMEM((CH, 128), jnp.float32),
        pltpu.VMEM_SHARED((N_PAD, 128), jnp.float32),
    ],
)


def _spmm_body(col_split, src_hbm, dst_hbm, table_hbm, zeros_hbm, out_hbm,
               idxs2, idxo2, idxd2, rows2, agg_sh, gsem, ssem0, ssem1,
               isem0, isem1):
    c = lax.axis_index("c")
    s = lax.axis_index("s")
    r0 = s * NR
    pltpu.sync_copy(zeros_hbm.at[pl.ds(r0, NR)], agg_sh.at[pl.ds(r0, NR)])
    plsc.subcore_barrier()
    ssem = [ssem0, ssem1]
    isem = [isem0, isem1]
    if col_split:
        chunks = CHUNKS_FULL
        ebase = s * (CHUNKS_FULL * CH)
        off = c * N_PAD
    else:
        chunks = CHUNKS_HALF
        ebase = (c * NT + s) * (CHUNKS_HALF * CH)
        off = None

    pltpu.async_copy(src_hbm.at[pl.ds(ebase, CH)], idxs2.at[0], isem[0])

    def window(w, carry):
        for b2 in range(2):
            k = 2 * w + b2
            b = ebase + k * CH

            @pl.when(k >= 2)
            def _():
                pltpu.make_async_copy(zeros_hbm.at[pl.ds(0, CH)],
                                      rows2.at[b2], ssem[b2]).wait()

            pltpu.make_async_copy(src_hbm.at[pl.ds(0, CH)], idxs2.at[b2],
                                  isem[b2]).wait()
            if off is not None:
                for j in range(CH // 16):
                    idxo2[b2, pl.ds(j * 16, 16)] = (
                        idxs2[b2, pl.ds(j * 16, 16)] + off)
                gidx = idxo2.at[b2]
            else:
                gidx = idxs2.at[b2]
            gather = pltpu.async_copy(table_hbm.at[gidx], rows2.at[b2], gsem)

            @pl.when(k + 1 < chunks)
            def _():
                pltpu.async_copy(src_hbm.at[pl.ds(b + CH, CH)],
                                 idxs2.at[1 - b2], isem[1 - b2])

            gather.wait()
            pltpu.sync_copy(dst_hbm.at[pl.ds(b, CH)], idxd2.at[b2])
            pltpu.async_copy(rows2.at[b2], agg_sh.at[idxd2.at[b2]],
                             ssem[b2], add=True)
        return carry

    lax.fori_loop(0, chunks // 2, window, 0)
    pltpu.make_async_copy(zeros_hbm.at[pl.ds(0, CH)], rows2.at[0],
                          ssem[0]).wait()
    pltpu.make_async_copy(zeros_hbm.at[pl.ds(0, CH)], rows2.at[1],
                          ssem[1]).wait()
    plsc.subcore_barrier()
    pltpu.sync_copy(agg_sh.at[pl.ds(r0, NR)],
                    out_hbm.at[pl.ds(c * N_PAD + r0, NR)])


def _make_spmm(col_split):
    return pl.kernel(
        lambda *args: _spmm_body(col_split, *args),
        out_type=jax.ShapeDtypeStruct((2 * N_PAD, 128), jnp.float32),
        mesh=_MESH,
        scratch_types=[
            pltpu.VMEM((2, CH), jnp.int32),
            pltpu.VMEM((2, CH), jnp.int32),
            pltpu.VMEM((2, CH), jnp.int32),
            pltpu.VMEM((2, CH, 128), jnp.float32),
            pltpu.VMEM_SHARED((N_PAD, 128), jnp.float32),
            pltpu.SemaphoreType.DMA,
            pltpu.SemaphoreType.DMA,
            pltpu.SemaphoreType.DMA,
            pltpu.SemaphoreType.DMA,
            pltpu.SemaphoreType.DMA,
        ],
    )


_spmm_l1 = _make_spmm(False)
_spmm_l2 = _make_spmm(True)


def _tc1_body(x_ref, dego_ref, xs_ref):
    sc = lax.rsqrt(jnp.maximum(dego_ref[:, 0:1], 1.0))
    xs_ref[...] = x_ref[...] * sc


_tc1_call = pl.pallas_call(
    _tc1_body,
    grid=(NB,),
    in_specs=[
        pl.BlockSpec((128, 128), lambda i: (i, 0)),
        pl.BlockSpec((128, 128), lambda i: (i, 0)),
    ],
    out_specs=pl.BlockSpec((128, 128), lambda i: (i, 0)),
    out_shape=jax.ShapeDtypeStruct((N_PAD, 128), jnp.float32),
)


def _tc2_body(agga_ref, aggb_ref, dego_ref, degi_ref, w_ref, b_ref, out_ref):
    a = agga_ref[...] + aggb_ref[...]
    t = lax.rsqrt(jnp.maximum(degi_ref[:, 0:1], 1.0))
    y = jnp.dot(t * a, w_ref[...], preferred_element_type=jnp.float32)
    y = y + b_ref[...]
    y = jnp.where(y > 0, y, 0.01 * y)
    sc = lax.rsqrt(jnp.maximum(dego_ref[:, 0:1], 1.0))
    out_ref[...] = sc * y


_tc2_call = pl.pallas_call(
    _tc2_body,
    grid=(2, NB),
    in_specs=[
        pl.BlockSpec((128, 128), lambda j, i: (i, 0)),
        pl.BlockSpec((128, 128), lambda j, i: (NB + i, 0)),
        pl.BlockSpec((128, 128), lambda j, i: (i, 0)),
        pl.BlockSpec((128, 128), lambda j, i: (NB + i, 0)),
        pl.BlockSpec((128, 128), lambda j, i: (0, j)),
        pl.BlockSpec((1, 128), lambda j, i: (0, j)),
    ],
    out_specs=pl.BlockSpec((128, 128), lambda j, i: (j * NB + i, 0)),
    out_shape=jax.ShapeDtypeStruct((2 * N_PAD, 128), jnp.float32),
)


def _tc3_body(agga_ref, aggb_ref, degi_ref, w_ref, b_ref, out_ref):
    a = jnp.concatenate([agga_ref[...], aggb_ref[...]], axis=1)
    t = lax.rsqrt(jnp.maximum(degi_ref[:, 0:1], 1.0))
    y = jnp.dot(t * a, w_ref[...], preferred_element_type=jnp.float32)
    out_ref[...] = y + b_ref[...]


_tc3_call = pl.pallas_call(
    _tc3_body,
    grid=(2, NB),
    in_specs=[
        pl.BlockSpec((128, 128), lambda j, i: (i, 0)),
        pl.BlockSpec((128, 128), lambda j, i: (NB + i, 0)),
        pl.BlockSpec((128, 128), lambda j, i: (NB + i, 0)),
        pl.BlockSpec((256, 128), lambda j, i: (0, j)),
        pl.BlockSpec((1, 128), lambda j, i: (0, j)),
    ],
    out_specs=pl.BlockSpec((128, 128), lambda j, i: (i, j)),
    out_shape=jax.ShapeDtypeStruct((N_PAD, DH), jnp.float32),
)


def kernel(n_feat, edge_index, W1, b1, W2, b2):
    f32 = jnp.float32
    x_pad = jnp.zeros((N_PAD, DIN), f32).at[:N].set(n_feat)
    pad_idx = N + jnp.arange(E_PAD, dtype=jnp.int32) % (N_PAD - N)
    src_pad = pad_idx.at[:E].set(edge_index[0])
    dst_pad = pad_idx.at[:E].set(edge_index[1])
    edges_flat = jnp.concatenate([src_pad, dst_pad])
    ones128 = jnp.ones((CH, 128), f32)
    zeros128 = jnp.zeros((N_PAD, 128), f32)

    degs = _deg_call(edges_flat, ones128, zeros128)
    xs1 = _tc1_call(x_pad, degs)
    agg1 = _spmm_l1(src_pad, dst_pad, xs1, zeros128)
    xs2 = _tc2_call(agg1, agg1, degs, degs, W1, b1.reshape(1, DH))
    agg2 = _spmm_l2(src_pad, dst_pad, xs2, zeros128)
    out = _tc3_call(agg2, agg2, degs, W2, b2.reshape(1, DH))
    return out[:N]
